# baseline (speedup 1.0000x reference)
import functools

import jax
import jax.numpy as jnp
from jax import lax
from jax.experimental import pallas as pl
from jax.experimental.pallas import tpu as pltpu
from jax.experimental.pallas import tpu_sc as plsc

N = 10000
E = 320000
NC, NS, L = 2, 16, 16
NW = NC * NS
EPW = E // NW
K = 80
NB = 8
PD = NB // 2
NCH = 125
NCHF = (NCH // NB) * NB
NPAD = 10240
RPS = NPAD // NS
ZR = 64

_mesh = plsc.VectorSubcoreMesh(core_axis_name="c", subcore_axis_name="s",
                               num_cores=NC, num_subcores=NS)
_sc_params = pltpu.CompilerParams(use_tc_tiling_on_sc=False)


def _fill(ref, nrows, ncols, value):
    v = jnp.full((L,), value, jnp.float32)

    @pl.loop(0, nrows)
    def _(r):
        @pl.loop(0, ncols, step=L)
        def _(c):
            ref.at[r, pl.ds(c, L)][...] = v


def _zero_acc(acc, zerov, f, base_row):
    _fill(zerov, ZR, f, 0.0)
    for z in range(RPS // ZR):
        pltpu.sync_copy(zerov, acc.at[pl.ds(base_row + z * ZR, ZR)])


def _sc_degree(dstp):
    @functools.partial(
        pl.kernel,
        out_type=jax.ShapeDtypeStruct((NC, NPAD, L), jnp.float32),
        mesh=_mesh,
        compiler_params=_sc_params,
        scratch_types=[
            pltpu.VMEM((NCH, K), jnp.int32),
            pltpu.VMEM((K, L), jnp.float32),
            pltpu.VMEM((ZR, L), jnp.float32),
            pltpu.VMEM_SHARED((NPAD, L), jnp.float32),
        ],
    )
    def deg_kernel(dstp_hbm, out_hbm, dstv, onesv, zerov, acc):
        cid = lax.axis_index("c")
        sid = lax.axis_index("s")
        wid = cid * NS + sid
        _fill(onesv, K, L, 1.0)
        base_row = sid * RPS
        _zero_acc(acc, zerov, L, base_row)
        pltpu.sync_copy(dstp_hbm.at[wid], dstv)
        plsc.subcore_barrier()

        @pl.loop(0, NCH)
        def _(c):
            pltpu.sync_copy(onesv, acc.at[dstv.at[c]], add=True)

        plsc.subcore_barrier()
        pltpu.sync_copy(acc.at[pl.ds(base_row, RPS)],
                        out_hbm.at[cid, pl.ds(base_row, RPS)])

    return deg_kernel(dstp)


def _sc_aggregate(g, srcp, dstp, f):
    @functools.partial(
        pl.kernel,
        out_type=jax.ShapeDtypeStruct((NC, NPAD, f), jnp.float32),
        mesh=_mesh,
        compiler_params=_sc_params,
        scratch_types=(
            [
                pltpu.VMEM((NCH, K), jnp.int32),
                pltpu.VMEM((NCH, K), jnp.int32),
            ]
            + [pltpu.VMEM((K, f), jnp.float32) for _ in range(NB)]
            + [
                pltpu.VMEM((ZR, f), jnp.float32),
                pltpu.VMEM_SHARED((NPAD, f), jnp.float32),
            ]
            + [pltpu.SemaphoreType.DMA for _ in range(2 * NB)]
        ),
    )
    def agg_kernel(g_hbm, srcp_hbm, dstp_hbm, out_hbm, srcv, dstv, *scr):
        cid = lax.axis_index("c")
        sid = lax.axis_index("s")
        wid = cid * NS + sid
        rows = list(scr[:NB])
        zerov = scr[NB]
        acc = scr[NB + 1]
        gsem = list(scr[NB + 2:2 * NB + 2])
        ssem = list(scr[2 * NB + 2:])
        base_row = sid * RPS
        _zero_acc(acc, zerov, f, base_row)
        pltpu.sync_copy(srcp_hbm.at[wid], srcv)
        pltpu.sync_copy(dstp_hbm.at[wid], dstv)
        plsc.subcore_barrier()

        def wait_gather(b):
            pltpu.make_async_copy(g_hbm.at[pl.ds(0, K)], rows[b], gsem[b]).wait()

        def wait_scatter(b):
            pltpu.make_async_copy(g_hbm.at[pl.ds(0, K)],
                                  acc.at[pl.ds(0, K)], ssem[b]).wait()

        def gather(c, b):
            pltpu.async_copy(g_hbm.at[srcv.at[c]], rows[b], gsem[b])

        def scatter(c, b):
            pltpu.async_copy(rows[b], acc.at[dstv.at[c]], ssem[b], add=True)

        for b in range(NB - PD):
            gather(b, b)

        for c in range(NB):
            wait_gather(c)
            scatter(c, c)
            b2 = (c + NB - PD) % NB
            if c >= PD:
                wait_scatter(b2)
            gather(c + NB - PD, b2)

        @pl.loop(NB, NCHF, step=NB)
        def _(i):
            for b in range(NB):
                c = i + b
                wait_gather(b)
                scatter(c, b)
                b2 = (b + NB - PD) % NB
                wait_scatter(b2)
                gather(c + NB - PD, b2)

        for t in range(NCH - NCHF):
            c = NCHF + t
            b = c % NB
            wait_gather(b)
            scatter(c, b)
            g2 = c + NB - PD
            if g2 < NCH:
                b2 = g2 % NB
                wait_scatter(b2)
                gather(g2, b2)

        for c in range(NCH - NB, NCH):
            wait_scatter(c % NB)

        plsc.subcore_barrier()
        pltpu.sync_copy(acc.at[pl.ds(base_row, RPS)],
                        out_hbm.at[cid, pl.ds(base_row, RPS)])

    return agg_kernel(g, srcp, dstp)


def _dinv(d_ref):
    return lax.rsqrt(1.0 + d_ref[0, 0:N, 0:1] + d_ref[1, 0:N, 0:1])


def _dot(a, b):
    return lax.dot_general(a, b, (((1,), (0,)), ((), ())),
                           preferred_element_type=jnp.float32,
                           precision=lax.Precision.HIGHEST)


def _tc_scale_mm(degp, x, w):
    def body(d_ref, x_ref, w_ref, o_ref):
        o_ref[0:N] = _dinv(d_ref) * _dot(x_ref[...], w_ref[...])
        o_ref[N:NPAD] = jnp.zeros((NPAD - N, w_ref.shape[1]), jnp.float32)

    return pl.pallas_call(
        body, out_shape=jax.ShapeDtypeStruct((NPAD, w.shape[1]), jnp.float32)
    )(degp, x, w)


def _tc_combine(degp, p, g, b, w):
    def body(d_ref, p_ref, g_ref, b_ref, w_ref, o_ref):
        dinv = _dinv(d_ref)
        t = dinv * (p_ref[0, 0:N] + p_ref[1, 0:N] + g_ref[0:N]) + b_ref[...]
        t = jnp.maximum(t, 0.0)
        o_ref[0:N] = dinv * _dot(t, w_ref[...])
        o_ref[N:NPAD] = jnp.zeros((NPAD - N, w_ref.shape[1]), jnp.float32)

    return pl.pallas_call(
        body,
        out_shape=jax.ShapeDtypeStruct((NPAD, w.shape[1]), jnp.float32),
    )(degp, p, g, b, w)


def _tc_final(degp, p, g, b):
    def body(d_ref, p_ref, g_ref, b_ref, o_ref):
        o_ref[...] = (_dinv(d_ref) * (p_ref[0, 0:N] + p_ref[1, 0:N] + g_ref[0:N])
                      + b_ref[...])

    return pl.pallas_call(
        body, out_shape=jax.ShapeDtypeStruct((N, g.shape[1]), jnp.float32)
    )(degp, p, g, b)


def _pad_idx(idx):
    return idx.reshape(NW, NCH, K)


def kernel(x, edge_index, W1, b1, W2, b2, W3, b3):
    srcp = _pad_idx(edge_index[0])
    dstp = _pad_idx(edge_index[1])

    degp = _sc_degree(dstp)
    g1 = _tc_scale_mm(degp, x, W1)
    p1 = _sc_aggregate(g1, srcp, dstp, 64)

    g2 = _tc_combine(degp, p1, g1, b1.reshape(1, -1), W2)
    p2 = _sc_aggregate(g2, srcp, dstp, 64)

    W3p = jnp.pad(W3, ((0, 0), (0, 8)))
    b3p = jnp.pad(b3, (0, 8))
    g3 = _tc_combine(degp, p2, g2, b2.reshape(1, -1), W3p)
    p3 = _sc_aggregate(g3, srcp, dstp, 48)

    out = _tc_final(degp, p3, g3, b3p.reshape(1, -1))
    return out[:, :40]


# speedup vs baseline: 1.0087x; 1.0087x over previous
import functools

import jax
import jax.numpy as jnp
from jax import lax
from jax.experimental import pallas as pl
from jax.experimental.pallas import tpu as pltpu
from jax.experimental.pallas import tpu_sc as plsc

N = 10000
E = 320000
NC, NS, L = 2, 16, 16
NW = NC * NS
EPW = E // NW
K = 80
NB = 8
PD = NB // 2
NCH = 125
NCHF = (NCH // NB) * NB
NPAD = 10240
RPS = NPAD // NS
ZR = 128

_mesh = plsc.VectorSubcoreMesh(core_axis_name="c", subcore_axis_name="s",
                               num_cores=NC, num_subcores=NS)
_sc_params = pltpu.CompilerParams(use_tc_tiling_on_sc=False)


def _fill(ref, nrows, ncols, value):
    v = jnp.full((L,), value, jnp.float32)

    @pl.loop(0, nrows)
    def _(r):
        @pl.loop(0, ncols, step=L)
        def _(c):
            ref.at[r, pl.ds(c, L)][...] = v


def _zero_acc(acc, zerov, f, base_row):
    _fill(zerov, ZR, f, 0.0)
    for z in range(RPS // ZR):
        pltpu.sync_copy(zerov, acc.at[pl.ds(base_row + z * ZR, ZR)])


def _sc_degree(dstp):
    @functools.partial(
        pl.kernel,
        out_type=jax.ShapeDtypeStruct((NC, NPAD, L), jnp.float32),
        mesh=_mesh,
        compiler_params=_sc_params,
        scratch_types=[
            pltpu.VMEM((NCH, K), jnp.int32),
            pltpu.VMEM((K, L), jnp.float32),
            pltpu.VMEM((ZR, L), jnp.float32),
            pltpu.VMEM_SHARED((NPAD, L), jnp.float32),
        ],
    )
    def deg_kernel(dstp_hbm, out_hbm, dstv, onesv, zerov, acc):
        cid = lax.axis_index("c")
        sid = lax.axis_index("s")
        wid = cid * NS + sid
        _fill(onesv, K, L, 1.0)
        base_row = sid * RPS
        _zero_acc(acc, zerov, L, base_row)
        pltpu.sync_copy(dstp_hbm.at[wid], dstv)
        plsc.subcore_barrier()

        @pl.loop(0, NCH)
        def _(c):
            pltpu.sync_copy(onesv, acc.at[dstv.at[c]], add=True)

        plsc.subcore_barrier()
        pltpu.sync_copy(acc.at[pl.ds(base_row, RPS)],
                        out_hbm.at[cid, pl.ds(base_row, RPS)])

    return deg_kernel(dstp)


def _sc_aggregate(g, srcp, dstp, f):
    @functools.partial(
        pl.kernel,
        out_type=jax.ShapeDtypeStruct((NC, NPAD, f), jnp.float32),
        mesh=_mesh,
        compiler_params=_sc_params,
        scratch_types=(
            [
                pltpu.VMEM((NCH, K), jnp.int32),
                pltpu.VMEM((NCH, K), jnp.int32),
            ]
            + [pltpu.VMEM((K, f), jnp.float32) for _ in range(NB)]
            + [
                pltpu.VMEM((ZR, f), jnp.float32),
                pltpu.VMEM_SHARED((NPAD, f), jnp.float32),
            ]
            + [pltpu.SemaphoreType.DMA for _ in range(2 * NB)]
        ),
    )
    def agg_kernel(g_hbm, srcp_hbm, dstp_hbm, out_hbm, srcv, dstv, *scr):
        cid = lax.axis_index("c")
        sid = lax.axis_index("s")
        wid = cid * NS + sid
        rows = list(scr[:NB])
        zerov = scr[NB]
        acc = scr[NB + 1]
        gsem = list(scr[NB + 2:2 * NB + 2])
        ssem = list(scr[2 * NB + 2:])
        base_row = sid * RPS
        _zero_acc(acc, zerov, f, base_row)
        pltpu.sync_copy(srcp_hbm.at[wid], srcv)
        pltpu.sync_copy(dstp_hbm.at[wid], dstv)
        plsc.subcore_barrier()

        def wait_gather(b):
            pltpu.make_async_copy(g_hbm.at[pl.ds(0, K)], rows[b], gsem[b]).wait()

        def wait_scatter(b):
            pltpu.make_async_copy(g_hbm.at[pl.ds(0, K)],
                                  acc.at[pl.ds(0, K)], ssem[b]).wait()

        def gather(c, b):
            pltpu.async_copy(g_hbm.at[srcv.at[c]], rows[b], gsem[b])

        def scatter(c, b):
            pltpu.async_copy(rows[b], acc.at[dstv.at[c]], ssem[b], add=True)

        for b in range(NB - PD):
            gather(b, b)

        for c in range(NB):
            wait_gather(c)
            scatter(c, c)
            b2 = (c + NB - PD) % NB
            if c >= PD:
                wait_scatter(b2)
            gather(c + NB - PD, b2)

        @pl.loop(NB, NCHF, step=NB)
        def _(i):
            for b in range(NB):
                c = i + b
                wait_gather(b)
                scatter(c, b)
                b2 = (b + NB - PD) % NB
                wait_scatter(b2)
                gather(c + NB - PD, b2)

        for t in range(NCH - NCHF):
            c = NCHF + t
            b = c % NB
            wait_gather(b)
            scatter(c, b)
            g2 = c + NB - PD
            if g2 < NCH:
                b2 = g2 % NB
                wait_scatter(b2)
                gather(g2, b2)

        for c in range(NCH - NB, NCH):
            wait_scatter(c % NB)

        plsc.subcore_barrier()
        pltpu.sync_copy(acc.at[pl.ds(base_row, RPS)],
                        out_hbm.at[cid, pl.ds(base_row, RPS)])

    return agg_kernel(g, srcp, dstp)


def _dinv(d_ref):
    return lax.rsqrt(1.0 + d_ref[0, 0:N, 0:1] + d_ref[1, 0:N, 0:1])


def _dot(a, b):
    return lax.dot_general(a, b, (((1,), (0,)), ((), ())),
                           preferred_element_type=jnp.float32,
                           precision=lax.Precision.HIGHEST)


def _tc_matmul(x, w):
    def body(x_ref, w_ref, o_ref):
        o_ref[...] = _dot(x_ref[...], w_ref[...])

    return pl.pallas_call(
        body,
        out_shape=jax.ShapeDtypeStruct((x.shape[0], w.shape[1]), jnp.float32),
    )(x, w)


def _tc_scale(degp, h):
    def body(d_ref, h_ref, o_ref):
        o_ref[0:N] = _dinv(d_ref) * h_ref[...]
        o_ref[N:NPAD] = jnp.zeros((NPAD - N, h_ref.shape[1]), jnp.float32)

    return pl.pallas_call(
        body, out_shape=jax.ShapeDtypeStruct((NPAD, h.shape[1]), jnp.float32)
    )(degp, h)


def _tc_combine(degp, p, g, b, w):
    def body(d_ref, p_ref, g_ref, b_ref, w_ref, o_ref):
        dinv = _dinv(d_ref)
        t = dinv * (p_ref[0, 0:N] + p_ref[1, 0:N] + g_ref[0:N]) + b_ref[...]
        t = jnp.maximum(t, 0.0)
        o_ref[0:N] = dinv * _dot(t, w_ref[...])
        o_ref[N:NPAD] = jnp.zeros((NPAD - N, w_ref.shape[1]), jnp.float32)

    return pl.pallas_call(
        body,
        out_shape=jax.ShapeDtypeStruct((NPAD, w.shape[1]), jnp.float32),
    )(degp, p, g, b, w)


def _tc_final(degp, p, g, b):
    def body(d_ref, p_ref, g_ref, b_ref, o_ref):
        o_ref[...] = (_dinv(d_ref) * (p_ref[0, 0:N] + p_ref[1, 0:N] + g_ref[0:N])
                      + b_ref[...])

    return pl.pallas_call(
        body, out_shape=jax.ShapeDtypeStruct((N, g.shape[1]), jnp.float32)
    )(degp, p, g, b)


def _pad_idx(idx):
    return idx.reshape(NW, NCH, K)


def kernel(x, edge_index, W1, b1, W2, b2, W3, b3):
    srcp = _pad_idx(edge_index[0])
    dstp = _pad_idx(edge_index[1])

    degp = _sc_degree(dstp)
    h1 = _tc_matmul(x, W1)
    g1 = _tc_scale(degp, h1)
    p1 = _sc_aggregate(g1, srcp, dstp, 64)

    g2 = _tc_combine(degp, p1, g1, b1.reshape(1, -1), W2)
    p2 = _sc_aggregate(g2, srcp, dstp, 64)

    W3p = jnp.pad(W3, ((0, 0), (0, 8)))
    b3p = jnp.pad(b3, (0, 8))
    g3 = _tc_combine(degp, p2, g2, b2.reshape(1, -1), W3p)
    p3 = _sc_aggregate(g3, srcp, dstp, 48)

    out = _tc_final(degp, p3, g3, b3p.reshape(1, -1))
    return out[:, :40]


# speedup vs baseline: 1.0309x; 1.0220x over previous
import functools

import jax
import jax.numpy as jnp
from jax import lax
from jax.experimental import pallas as pl
from jax.experimental.pallas import tpu as pltpu
from jax.experimental.pallas import tpu_sc as plsc

N = 10000
E = 320000
NC, NS, L = 2, 16, 16
NW = NC * NS
EPW = E // NW
K = 80
NB = 10
PD = NB // 2
NCH = 125
NCHF = (NCH // NB) * NB
NPAD = 10240
RPS = NPAD // NS
ZR = 128

_mesh = plsc.VectorSubcoreMesh(core_axis_name="c", subcore_axis_name="s",
                               num_cores=NC, num_subcores=NS)
_sc_params = pltpu.CompilerParams(use_tc_tiling_on_sc=False)


def _fill(ref, nrows, ncols, value):
    v = jnp.full((L,), value, jnp.float32)

    @pl.loop(0, nrows)
    def _(r):
        @pl.loop(0, ncols, step=L)
        def _(c):
            ref.at[r, pl.ds(c, L)][...] = v


def _zero_acc(acc, zerov, f, base_row):
    _fill(zerov, ZR, f, 0.0)
    for z in range(RPS // ZR):
        pltpu.sync_copy(zerov, acc.at[pl.ds(base_row + z * ZR, ZR)])


def _sc_degree(dstp):
    @functools.partial(
        pl.kernel,
        out_type=jax.ShapeDtypeStruct((NC, NPAD, L), jnp.float32),
        mesh=_mesh,
        compiler_params=_sc_params,
        scratch_types=[
            pltpu.VMEM((NCH, K), jnp.int32),
            pltpu.VMEM((K, L), jnp.float32),
            pltpu.VMEM((ZR, L), jnp.float32),
            pltpu.VMEM_SHARED((NPAD, L), jnp.float32),
        ],
    )
    def deg_kernel(dstp_hbm, out_hbm, dstv, onesv, zerov, acc):
        cid = lax.axis_index("c")
        sid = lax.axis_index("s")
        wid = cid * NS + sid
        _fill(onesv, K, L, 1.0)
        base_row = sid * RPS
        _zero_acc(acc, zerov, L, base_row)
        pltpu.sync_copy(dstp_hbm.at[wid], dstv)
        plsc.subcore_barrier()

        @pl.loop(0, NCH)
        def _(c):
            pltpu.sync_copy(onesv, acc.at[dstv.at[c]], add=True)

        plsc.subcore_barrier()
        pltpu.sync_copy(acc.at[pl.ds(base_row, RPS)],
                        out_hbm.at[cid, pl.ds(base_row, RPS)])

    return deg_kernel(dstp)


def _sc_aggregate(g, srcp, dstp, f):
    @functools.partial(
        pl.kernel,
        out_type=jax.ShapeDtypeStruct((NC, NPAD, f), jnp.float32),
        mesh=_mesh,
        compiler_params=_sc_params,
        scratch_types=(
            [
                pltpu.VMEM((NCH, K), jnp.int32),
                pltpu.VMEM((NCH, K), jnp.int32),
            ]
            + [pltpu.VMEM((K, f), jnp.float32) for _ in range(NB)]
            + [
                pltpu.VMEM((ZR, f), jnp.float32),
                pltpu.VMEM_SHARED((NPAD, f), jnp.float32),
            ]
            + [pltpu.SemaphoreType.DMA for _ in range(2 * NB)]
        ),
    )
    def agg_kernel(g_hbm, srcp_hbm, dstp_hbm, out_hbm, srcv, dstv, *scr):
        cid = lax.axis_index("c")
        sid = lax.axis_index("s")
        wid = cid * NS + sid
        rows = list(scr[:NB])
        zerov = scr[NB]
        acc = scr[NB + 1]
        gsem = list(scr[NB + 2:2 * NB + 2])
        ssem = list(scr[2 * NB + 2:])
        base_row = sid * RPS
        _zero_acc(acc, zerov, f, base_row)
        pltpu.sync_copy(srcp_hbm.at[wid], srcv)
        pltpu.sync_copy(dstp_hbm.at[wid], dstv)
        plsc.subcore_barrier()

        def wait_gather(b):
            pltpu.make_async_copy(g_hbm.at[pl.ds(0, K)], rows[b], gsem[b]).wait()

        def wait_scatter(b):
            pltpu.make_async_copy(g_hbm.at[pl.ds(0, K)],
                                  acc.at[pl.ds(0, K)], ssem[b]).wait()

        def gather(c, b):
            pltpu.async_copy(g_hbm.at[srcv.at[c]], rows[b], gsem[b])

        def scatter(c, b):
            pltpu.async_copy(rows[b], acc.at[dstv.at[c]], ssem[b], add=True)

        for b in range(NB - PD):
            gather(b, b)

        for c in range(NB):
            wait_gather(c)
            scatter(c, c)
            b2 = (c + NB - PD) % NB
            if c >= PD:
                wait_scatter(b2)
            gather(c + NB - PD, b2)

        @pl.loop(NB, NCHF, step=NB)
        def _(i):
            for b in range(NB):
                c = i + b
                wait_gather(b)
                scatter(c, b)
                b2 = (b + NB - PD) % NB
                wait_scatter(b2)
                gather(c + NB - PD, b2)

        for t in range(NCH - NCHF):
            c = NCHF + t
            b = c % NB
            wait_gather(b)
            scatter(c, b)
            g2 = c + NB - PD
            if g2 < NCH:
                b2 = g2 % NB
                wait_scatter(b2)
                gather(g2, b2)

        for c in range(NCH - NB, NCH):
            wait_scatter(c % NB)

        plsc.subcore_barrier()
        pltpu.sync_copy(acc.at[pl.ds(base_row, RPS)],
                        out_hbm.at[cid, pl.ds(base_row, RPS)])

    return agg_kernel(g, srcp, dstp)


def _dinv(d_ref):
    return lax.rsqrt(1.0 + d_ref[0, 0:N, 0:1] + d_ref[1, 0:N, 0:1])


def _dot(a, b):
    return lax.dot_general(a, b, (((1,), (0,)), ((), ())),
                           preferred_element_type=jnp.float32,
                           precision=lax.Precision.HIGHEST)


def _tc_matmul(x, w):
    def body(x_ref, w_ref, o_ref):
        o_ref[...] = _dot(x_ref[...], w_ref[...])

    return pl.pallas_call(
        body,
        out_shape=jax.ShapeDtypeStruct((x.shape[0], w.shape[1]), jnp.float32),
    )(x, w)


def _tc_scale(degp, h):
    def body(d_ref, h_ref, o_ref):
        o_ref[0:N] = _dinv(d_ref) * h_ref[...]
        o_ref[N:NPAD] = jnp.zeros((NPAD - N, h_ref.shape[1]), jnp.float32)

    return pl.pallas_call(
        body, out_shape=jax.ShapeDtypeStruct((NPAD, h.shape[1]), jnp.float32)
    )(degp, h)


def _tc_combine(degp, p, g, b, w):
    def body(d_ref, p_ref, g_ref, b_ref, w_ref, o_ref):
        dinv = _dinv(d_ref)
        t = dinv * (p_ref[0, 0:N] + p_ref[1, 0:N] + g_ref[0:N]) + b_ref[...]
        t = jnp.maximum(t, 0.0)
        o_ref[0:N] = dinv * _dot(t, w_ref[...])
        o_ref[N:NPAD] = jnp.zeros((NPAD - N, w_ref.shape[1]), jnp.float32)

    return pl.pallas_call(
        body,
        out_shape=jax.ShapeDtypeStruct((NPAD, w.shape[1]), jnp.float32),
    )(degp, p, g, b, w)


def _tc_final(degp, p, g, b):
    def body(d_ref, p_ref, g_ref, b_ref, o_ref):
        o_ref[...] = (_dinv(d_ref) * (p_ref[0, 0:N] + p_ref[1, 0:N] + g_ref[0:N])
                      + b_ref[...])

    return pl.pallas_call(
        body, out_shape=jax.ShapeDtypeStruct((N, g.shape[1]), jnp.float32)
    )(degp, p, g, b)


def _pad_idx(idx):
    return idx.reshape(NW, NCH, K)


def kernel(x, edge_index, W1, b1, W2, b2, W3, b3):
    srcp = _pad_idx(edge_index[0])
    dstp = _pad_idx(edge_index[1])

    degp = _sc_degree(dstp)
    h1 = _tc_matmul(x, W1)
    g1 = _tc_scale(degp, h1)
    p1 = _sc_aggregate(g1, srcp, dstp, 64)

    g2 = _tc_combine(degp, p1, g1, b1.reshape(1, -1), W2)
    p2 = _sc_aggregate(g2, srcp, dstp, 64)

    W3p = jnp.pad(W3, ((0, 0), (0, 8)))
    b3p = jnp.pad(b3, (0, 8))
    g3 = _tc_combine(degp, p2, g2, b2.reshape(1, -1), W3p)
    p3 = _sc_aggregate(g3, srcp, dstp, 48)

    out = _tc_final(degp, p3, g3, b3p.reshape(1, -1))
    return out[:, :40]
